# initial kernel scaffold (unmeasured)
import jax
import jax.numpy as jnp
from jax import lax
from jax.experimental import pallas as pl
from jax.experimental.pallas import tpu as pltpu

N_Z = 4
SCALE = 64 ** -0.5


def _partials_body(q_ref, k_ref, v_ref, m_ref, l_ref, o_ref):
    q = q_ref[0, 0]
    k = k_ref[0]
    v = v_ref[0]
    s = jnp.sum(k * q[None], axis=-1) * SCALE
    m = jnp.max(s, axis=0)
    p = jnp.exp(s - m[None, :])
    l = jnp.sum(p, axis=0)
    o = jnp.sum(p[:, :, None] * v, axis=0)
    m_ref[0, :] = m
    l_ref[0, :] = l
    o_ref[0] = o


def _local_partials(Q, K, V):
    b, kk, h, d = K.shape
    return pl.pallas_call(
        _partials_body,
        grid=(b,),
        in_specs=[
            pl.BlockSpec((1, 1, h, d), lambda i: (i, 0, 0, 0)),
            pl.BlockSpec((1, kk, h, d), lambda i: (i, 0, 0, 0)),
            pl.BlockSpec((1, kk, h, d), lambda i: (i, 0, 0, 0)),
        ],
        out_specs=[
            pl.BlockSpec((1, h), lambda i: (i, 0)),
            pl.BlockSpec((1, h), lambda i: (i, 0)),
            pl.BlockSpec((1, h, d), lambda i: (i, 0, 0)),
        ],
        out_shape=[
            jax.ShapeDtypeStruct((b, h), jnp.float32),
            jax.ShapeDtypeStruct((b, h), jnp.float32),
            jax.ShapeDtypeStruct((b, h, d), jnp.float32),
        ],
    )(Q, K, V)


def _combine_body(m_ref, l_ref, o_ref, out_ref,
                  cm_ref, cl_ref, co_ref, send_sems, recv_sems):
    my_x = lax.axis_index("x")
    my_y = lax.axis_index("y")
    my_z = lax.axis_index("z")

    tensors = ((m_ref, cm_ref), (l_ref, cl_ref), (o_ref, co_ref))

    cm_ref[pl.ds(my_z, 1)] = m_ref[...][None]
    cl_ref[pl.ds(my_z, 1)] = l_ref[...][None]
    co_ref[pl.ds(my_z, 1)] = o_ref[...][None]

    sends = []
    for dz in range(1, N_Z):
        zz = lax.rem(my_z + dz, N_Z)
        for ti, (src, dst) in enumerate(tensors):
            rdma = pltpu.make_async_remote_copy(
                src_ref=src,
                dst_ref=dst.at[my_z],
                send_sem=send_sems.at[dz - 1, ti],
                recv_sem=recv_sems.at[my_z, ti],
                device_id=(my_x, my_y, zz),
                device_id_type=pl.DeviceIdType.MESH,
            )
            rdma.start()
            sends.append(rdma)

    for dz in range(1, N_Z):
        src_z = lax.rem(my_z + dz, N_Z)
        for ti, (src, dst) in enumerate(tensors):
            rdma = pltpu.make_async_remote_copy(
                src_ref=src,
                dst_ref=dst.at[src_z],
                send_sem=send_sems.at[dz - 1, ti],
                recv_sem=recv_sems.at[src_z, ti],
                device_id=(my_x, my_y, src_z),
                device_id_type=pl.DeviceIdType.MESH,
            )
            rdma.wait_recv()

    cm = cm_ref[...]
    cl = cl_ref[...]
    co = co_ref[...]
    g_m = jnp.max(cm, axis=0)
    alpha = jnp.exp(cm - g_m[None])
    g_l = jnp.sum(cl * alpha, axis=0)
    o = jnp.sum(co * alpha[..., None], axis=0) / g_l[..., None]
    out_ref[...] = o[:, None]

    for rdma in sends:
        rdma.wait_send()


def _combine(m, l, o):
    b, h = m.shape
    d = o.shape[-1]
    return pl.pallas_call(
        _combine_body,
        in_specs=[
            pl.BlockSpec(memory_space=pltpu.VMEM),
            pl.BlockSpec(memory_space=pltpu.VMEM),
            pl.BlockSpec(memory_space=pltpu.VMEM),
        ],
        out_specs=pl.BlockSpec(memory_space=pltpu.VMEM),
        out_shape=jax.ShapeDtypeStruct((b, 1, h, d), jnp.float32),
        scratch_shapes=[
            pltpu.VMEM((N_Z, b, h), jnp.float32),
            pltpu.VMEM((N_Z, b, h), jnp.float32),
            pltpu.VMEM((N_Z, b, h, d), jnp.float32),
            pltpu.SemaphoreType.DMA((N_Z - 1, 3)),
            pltpu.SemaphoreType.DMA((N_Z, 3)),
        ],
        compiler_params=pltpu.CompilerParams(has_side_effects=True),
    )(m, l, o)


def kernel(Q, K, V):
    m, l, o = _local_partials(Q, K, V)
    return _combine(m, l, o)


# baseline (device time: 343659 ns/iter reference)
import jax
import jax.numpy as jnp
from jax import lax
from jax.experimental import pallas as pl
from jax.experimental.pallas import tpu as pltpu

N_Z = 4
SCALE = 64 ** -0.5


def _partials_body(q_ref, k_ref, v_ref, m_ref, l_ref, o_ref):
    q = q_ref[0, 0]
    k = k_ref[0]
    v = v_ref[0]
    s = jnp.sum(k * q[None], axis=-1) * SCALE
    m = jnp.max(s, axis=0)
    p = jnp.exp(s - m[None, :])
    l = jnp.sum(p, axis=0)
    o = jnp.sum(p[:, :, None] * v, axis=0)
    m_ref[0, 0, :] = m
    l_ref[0, 0, :] = l
    o_ref[0] = o


def _local_partials(Q, K, V):
    b, kk, h, d = K.shape
    return pl.pallas_call(
        _partials_body,
        grid=(b,),
        in_specs=[
            pl.BlockSpec((1, 1, h, d), lambda i: (i, 0, 0, 0)),
            pl.BlockSpec((1, kk, h, d), lambda i: (i, 0, 0, 0)),
            pl.BlockSpec((1, kk, h, d), lambda i: (i, 0, 0, 0)),
        ],
        out_specs=[
            pl.BlockSpec((1, 1, h), lambda i: (i, 0, 0)),
            pl.BlockSpec((1, 1, h), lambda i: (i, 0, 0)),
            pl.BlockSpec((1, h, d), lambda i: (i, 0, 0)),
        ],
        out_shape=[
            jax.ShapeDtypeStruct((b, 1, h), jnp.float32),
            jax.ShapeDtypeStruct((b, 1, h), jnp.float32),
            jax.ShapeDtypeStruct((b, h, d), jnp.float32),
        ],
        compiler_params=pltpu.CompilerParams(
            vmem_limit_bytes=100 * 1024 * 1024,
        ),
    )(Q, K, V)


def _combine_body(m_ref, l_ref, o_ref, out_ref,
                  cm_ref, cl_ref, co_ref, send_sems, recv_sems):
    my_x = lax.axis_index("x")
    my_y = lax.axis_index("y")
    my_z = lax.axis_index("z")

    tensors = ((m_ref, cm_ref), (l_ref, cl_ref), (o_ref, co_ref))

    cm_ref[pl.ds(my_z, 1)] = m_ref[...][None]
    cl_ref[pl.ds(my_z, 1)] = l_ref[...][None]
    co_ref[pl.ds(my_z, 1)] = o_ref[...][None]

    sends = []
    for dz in range(1, N_Z):
        zz = lax.rem(my_z + dz, N_Z)
        for ti, (src, dst) in enumerate(tensors):
            rdma = pltpu.make_async_remote_copy(
                src_ref=src,
                dst_ref=dst.at[my_z],
                send_sem=send_sems.at[dz - 1, ti],
                recv_sem=recv_sems.at[my_z, ti],
                device_id=(my_x, my_y, zz),
                device_id_type=pl.DeviceIdType.MESH,
            )
            rdma.start()
            sends.append(rdma)

    for dz in range(1, N_Z):
        src_z = lax.rem(my_z + dz, N_Z)
        for ti, (src, dst) in enumerate(tensors):
            rdma = pltpu.make_async_remote_copy(
                src_ref=src,
                dst_ref=dst.at[src_z],
                send_sem=send_sems.at[dz - 1, ti],
                recv_sem=recv_sems.at[src_z, ti],
                device_id=(my_x, my_y, src_z),
                device_id_type=pl.DeviceIdType.MESH,
            )
            rdma.wait_recv()

    cm = cm_ref[...][:, :, 0, :]
    cl = cl_ref[...][:, :, 0, :]
    co = co_ref[...]
    g_m = jnp.max(cm, axis=0)
    alpha = jnp.exp(cm - g_m[None])
    g_l = jnp.sum(cl * alpha, axis=0)
    o = jnp.sum(co * alpha[..., None], axis=0) / g_l[..., None]
    out_ref[...] = o[:, None]

    for rdma in sends:
        rdma.wait_send()


def _combine(m, l, o):
    b, h = m.shape[0], m.shape[-1]
    d = o.shape[-1]
    return pl.pallas_call(
        _combine_body,
        in_specs=[
            pl.BlockSpec(memory_space=pltpu.VMEM),
            pl.BlockSpec(memory_space=pltpu.VMEM),
            pl.BlockSpec(memory_space=pltpu.VMEM),
        ],
        out_specs=pl.BlockSpec(memory_space=pltpu.VMEM),
        out_shape=jax.ShapeDtypeStruct((b, 1, h, d), jnp.float32),
        scratch_shapes=[
            pltpu.VMEM((N_Z, b, 1, h), jnp.float32),
            pltpu.VMEM((N_Z, b, 1, h), jnp.float32),
            pltpu.VMEM((N_Z, b, h, d), jnp.float32),
            pltpu.SemaphoreType.DMA((N_Z - 1, 3)),
            pltpu.SemaphoreType.DMA((N_Z, 3)),
        ],
        compiler_params=pltpu.CompilerParams(has_side_effects=True),
    )(m, l, o)


def kernel(Q, K, V):
    m, l, o = _local_partials(Q, K, V)
    return _combine(m, l, o)


# device time: 196278 ns/iter; 1.7509x vs baseline; 1.7509x over previous
import jax
import jax.numpy as jnp
from jax import lax
from jax.experimental import pallas as pl
from jax.experimental.pallas import tpu as pltpu

N_Z = 4
SCALE = 64 ** -0.5


def _partials_body(q_ref, k_ref, v_ref, m_ref, l_ref, o_ref, *, h, d):
    kk = k_ref.shape[1]
    q = q_ref[0, 0]
    k2 = k_ref[0]
    v2 = v_ref[0]

    qcol = (q * SCALE).reshape(h * d, 1)
    row_h = lax.broadcasted_iota(jnp.int32, (h * d, h), 0) // d
    col_h = lax.broadcasted_iota(jnp.int32, (h * d, h), 1)
    q2 = jnp.where(row_h == col_h, qcol, 0.0).astype(jnp.bfloat16)

    s = jnp.dot(k2.astype(jnp.bfloat16), q2,
                preferred_element_type=jnp.float32)
    m = jnp.max(s, axis=0)
    p = jnp.exp(s - m[None, :])
    l = jnp.sum(p, axis=0)

    pt = p.T.astype(jnp.bfloat16)
    o2 = jnp.dot(pt, v2.astype(jnp.bfloat16),
                 preferred_element_type=jnp.float32)
    o3 = o2.reshape(h, h, d)
    sel = (lax.broadcasted_iota(jnp.int32, (h, h, d), 0)
           == lax.broadcasted_iota(jnp.int32, (h, h, d), 1))
    o = jnp.sum(jnp.where(sel, o3, 0.0), axis=0)

    m_ref[0, 0, :] = m
    l_ref[0, 0, :] = l
    o_ref[0] = o


def _local_partials(Q, K, V):
    b, kk, h, d = K.shape
    import functools
    Q2 = Q.reshape(b, 1, h * d)
    K2 = K.reshape(b, kk, h * d)
    V2 = V.reshape(b, kk, h * d)
    return pl.pallas_call(
        functools.partial(_partials_body, h=h, d=d),
        grid=(b,),
        in_specs=[
            pl.BlockSpec((1, 1, h * d), lambda i: (i, 0, 0)),
            pl.BlockSpec((1, kk, h * d), lambda i: (i, 0, 0)),
            pl.BlockSpec((1, kk, h * d), lambda i: (i, 0, 0)),
        ],
        out_specs=[
            pl.BlockSpec((1, 1, h), lambda i: (i, 0, 0)),
            pl.BlockSpec((1, 1, h), lambda i: (i, 0, 0)),
            pl.BlockSpec((1, h, d), lambda i: (i, 0, 0)),
        ],
        out_shape=[
            jax.ShapeDtypeStruct((b, 1, h), jnp.float32),
            jax.ShapeDtypeStruct((b, 1, h), jnp.float32),
            jax.ShapeDtypeStruct((b, h, d), jnp.float32),
        ],
        compiler_params=pltpu.CompilerParams(
            vmem_limit_bytes=100 * 1024 * 1024,
        ),
    )(Q2, K2, V2)


def _combine_body(m_ref, l_ref, o_ref, out_ref,
                  cm_ref, cl_ref, co_ref, send_sems, recv_sems):
    my_x = lax.axis_index("x")
    my_y = lax.axis_index("y")
    my_z = lax.axis_index("z")

    tensors = ((m_ref, cm_ref), (l_ref, cl_ref), (o_ref, co_ref))

    cm_ref[pl.ds(my_z, 1)] = m_ref[...][None]
    cl_ref[pl.ds(my_z, 1)] = l_ref[...][None]
    co_ref[pl.ds(my_z, 1)] = o_ref[...][None]

    sends = []
    for dz in range(1, N_Z):
        zz = lax.rem(my_z + dz, N_Z)
        for ti, (src, dst) in enumerate(tensors):
            rdma = pltpu.make_async_remote_copy(
                src_ref=src,
                dst_ref=dst.at[my_z],
                send_sem=send_sems.at[dz - 1, ti],
                recv_sem=recv_sems.at[my_z, ti],
                device_id=(my_x, my_y, zz),
                device_id_type=pl.DeviceIdType.MESH,
            )
            rdma.start()
            sends.append(rdma)

    for dz in range(1, N_Z):
        src_z = lax.rem(my_z + dz, N_Z)
        for ti, (src, dst) in enumerate(tensors):
            rdma = pltpu.make_async_remote_copy(
                src_ref=src,
                dst_ref=dst.at[src_z],
                send_sem=send_sems.at[dz - 1, ti],
                recv_sem=recv_sems.at[src_z, ti],
                device_id=(my_x, my_y, src_z),
                device_id_type=pl.DeviceIdType.MESH,
            )
            rdma.wait_recv()

    cm = cm_ref[...][:, :, 0, :]
    cl = cl_ref[...][:, :, 0, :]
    co = co_ref[...]
    g_m = jnp.max(cm, axis=0)
    alpha = jnp.exp(cm - g_m[None])
    g_l = jnp.sum(cl * alpha, axis=0)
    o = jnp.sum(co * alpha[..., None], axis=0) / g_l[..., None]
    out_ref[...] = o[:, None]

    for rdma in sends:
        rdma.wait_send()


def _combine(m, l, o):
    b, h = m.shape[0], m.shape[-1]
    d = o.shape[-1]
    return pl.pallas_call(
        _combine_body,
        in_specs=[
            pl.BlockSpec(memory_space=pltpu.VMEM),
            pl.BlockSpec(memory_space=pltpu.VMEM),
            pl.BlockSpec(memory_space=pltpu.VMEM),
        ],
        out_specs=pl.BlockSpec(memory_space=pltpu.VMEM),
        out_shape=jax.ShapeDtypeStruct((b, 1, h, d), jnp.float32),
        scratch_shapes=[
            pltpu.VMEM((N_Z, b, 1, h), jnp.float32),
            pltpu.VMEM((N_Z, b, 1, h), jnp.float32),
            pltpu.VMEM((N_Z, b, h, d), jnp.float32),
            pltpu.SemaphoreType.DMA((N_Z - 1, 3)),
            pltpu.SemaphoreType.DMA((N_Z, 3)),
        ],
        compiler_params=pltpu.CompilerParams(has_side_effects=True),
    )(m, l, o)


def kernel(Q, K, V):
    m, l, o = _local_partials(Q, K, V)
    return _combine(m, l, o)
